# baseline (speedup 1.0000x reference)
import functools
from math import factorial

import jax
import jax.numpy as jnp
from jax.experimental import pallas as pl
from jax.experimental.pallas import tpu as pltpu

_LMAX = 4
_NT = 8
_NP = _NT * _NT
_B = 1024
_BT = 256
_NB = _B // _BT

_LLLS = sorted([(l, l1, l2)
                for l1 in range(_LMAX + 1)
                for l2 in range(l1 + 1)
                for l in range(l1 - l2, min(l1 + l2, _LMAX) + 1)])
_TRIPLES_OF = [[(l1, l2) for (l, l1, l2) in _LLLS if l == lo]
               for lo in range(_LMAX + 1)]
_T_FF = [_NP * len(_TRIPLES_OF[lo]) for lo in range(_LMAX + 1)]
_CTOT = sum(_T_FF)
_CH_OFF = [sum(_T_FF[:l]) for l in range(_LMAX + 1)]
_LMOFF = [sum(2 * ll + 1 for ll in range(l)) for l in range(_LMAX + 1)]
_MTOT = _LMOFF[-1] + (2 * _LMAX + 1)
_AEOFF = [_NP * _LMOFF[l] for l in range(_LMAX + 1)]
_AEROWS = _NP * _MTOT


def _cgc(l1, m1, l2, m2, l, m):
    if m1 + m2 != m:
        return 0.0
    if l < abs(l1 - l2) or l > l1 + l2:
        return 0.0
    if abs(m1) > l1 or abs(m2) > l2 or abs(m) > l:
        return 0.0
    f = factorial
    pref = ((2 * l + 1) * f(l + l1 - l2) * f(l - l1 + l2) * f(l1 + l2 - l)
            / f(l1 + l2 + l + 1)) ** 0.5
    pref *= (f(l + m) * f(l - m) * f(l1 - m1) * f(l1 + m1)
             * f(l2 - m2) * f(l2 + m2)) ** 0.5
    s = 0.0
    kmin = max(0, l2 - l - m1, l1 + m2 - l)
    kmax = min(l1 + l2 - l, l1 - m1, l2 + m2)
    for k in range(kmin, kmax + 1):
        s += (-1.0) ** k / (f(k) * f(l1 + l2 - l - k) * f(l1 - m1 - k)
                            * f(l2 + m2 - k) * f(l - l2 + m1 + k)
                            * f(l - l1 - m2 + k))
    return pref * s


_PAIRS = sorted(set((l1, l2) for (_, l1, l2) in _LLLS))
_PLAN = {}
for (l1, l2) in _PAIRS:
    targets = {}
    for i1 in range(2 * l1 + 1):
        for i2 in range(2 * l2 + 1):
            m1, m2 = i1 - l1, i2 - l2
            tl = []
            for l in range(abs(l1 - l2), min(l1 + l2, _LMAX) + 1):
                if abs(m1 + m2) > l:
                    continue
                c = _cgc(l1, m1, l2, m2, l, m1 + m2)
                if c != 0.0:
                    t_idx = _TRIPLES_OF[l].index((l1, l2))
                    tl.append((l, t_idx, m1 + m2 + l, float(c)))
            if tl:
                targets[(i1, i2)] = tl
    _PLAN[(l1, l2)] = targets


def _compute_g(aer, aei, ber, bei):
    ar = {}
    ai = {}
    br = {}
    bi = {}
    for l in range(_LMAX + 1):
        for i in range(2 * l + 1):
            r0 = _AEOFF[l] + i * _NP
            ar[(l, i)] = aer[r0:r0 + _NP, :]
            ai[(l, i)] = aei[r0:r0 + _NP, :]
            br[(l, i)] = ber[r0:r0 + _NP, :]
            bi[(l, i)] = bei[r0:r0 + _NP, :]
    gr = {}
    gi = {}
    for (l1, l2) in _PAIRS:
        for (i1, i2), tl in _PLAN[(l1, l2)].items():
            xr, xi = ar[(l1, i1)], ai[(l1, i1)]
            yr, yi = br[(l2, i2)], bi[(l2, i2)]
            pr = xr * yr - xi * yi
            pi = xr * yi + xi * yr
            for (l, t_idx, m_idx, c) in tl:
                key = (l, t_idx, m_idx)
                if key in gr:
                    gr[key] = gr[key] + c * pr
                    gi[key] = gi[key] + c * pi
                else:
                    gr[key] = c * pr
                    gi[key] = c * pi
    zero = None
    for l in range(_LMAX + 1):
        for t in range(len(_TRIPLES_OF[l])):
            for m in range(2 * l + 1):
                if (l, t, m) not in gr:
                    if zero is None:
                        zero = jnp.zeros((_NP, _BT), jnp.float32)
                    gr[(l, t, m)] = zero
                    gi[(l, t, m)] = zero
    return gr, gi


def _cgbn_body(aer, aei, ber, bei, wcat, outr, outi, acc, wp):
    p = pl.program_id(0)
    j = pl.program_id(1)

    @pl.when(jnp.logical_and(p == 0, j == 0))
    def _():
        acc[...] = jnp.zeros_like(acc)

    gr, gi = _compute_g(aer[...], aei[...], ber[...], bei[...])

    @pl.when(p == 0)
    def _():
        for l in range(_LMAX + 1):
            for t in range(len(_TRIPLES_OF[l])):
                sq = None
                for m in range(2 * l + 1):
                    s = gr[(l, t, m)] * gr[(l, t, m)] \
                        + gi[(l, t, m)] * gi[(l, t, m)]
                    sq = s if sq is None else sq + s
                r0 = _CH_OFF[l] + t * _NP
                acc[r0:r0 + _NP, :] = acc[r0:r0 + _NP, :] + sq

    @pl.when(jnp.logical_and(p == 1, j == 0))
    def _():
        pw = jnp.sum(acc[...], axis=1, keepdims=True) * (1.0 / _B)
        wp[...] = wcat[...] * jax.lax.rsqrt(pw + 1e-5)

    @pl.when(p == 1)
    def _():
        for l in range(_LMAX + 1):
            nm = 2 * l + 1
            gmat_r = jnp.concatenate(
                [jnp.concatenate([gr[(l, t, m)] for m in range(nm)], axis=1)
                 for t in range(len(_TRIPLES_OF[l]))], axis=0)
            gmat_i = jnp.concatenate(
                [jnp.concatenate([gi[(l, t, m)] for m in range(nm)], axis=1)
                 for t in range(len(_TRIPLES_OF[l]))], axis=0)
            wl = wp[_CH_OFF[l]:_CH_OFF[l] + _T_FF[l], :]
            dn = (((0,), (0,)), ((), ()))
            o_r = jax.lax.dot_general(wl, gmat_r, dn,
                                      preferred_element_type=jnp.float32)
            o_i = jax.lax.dot_general(wl, gmat_i, dn,
                                      preferred_element_type=jnp.float32)
            for m in range(nm):
                outr[:, _LMOFF[l] + m, :] = o_r[:, m * _BT:(m + 1) * _BT]
                outi[:, _LMOFF[l] + m, :] = o_i[:, m * _BT:(m + 1) * _BT]


@functools.partial(jax.jit, static_argnums=())
def kernel(activations, W0, W1, W2, W3, W4):
    fr = activations[..., 0].T
    fi = activations[..., 1].T
    aer_l, aei_l, ber_l, bei_l = [], [], [], []
    off = 0
    for l in range(_LMAX + 1):
        n = _NT * (2 * l + 1)
        blk_r = fr[off:off + n, :].reshape(_NT, 2 * l + 1, _B)
        blk_i = fi[off:off + n, :].reshape(_NT, 2 * l + 1, _B)
        off += n
        ae_r = jnp.broadcast_to(blk_r[:, None], (_NT, _NT, 2 * l + 1, _B))
        ae_i = jnp.broadcast_to(blk_i[:, None], (_NT, _NT, 2 * l + 1, _B))
        be_r = jnp.broadcast_to(blk_r[None, :], (_NT, _NT, 2 * l + 1, _B))
        be_i = jnp.broadcast_to(blk_i[None, :], (_NT, _NT, 2 * l + 1, _B))
        for src, dst in ((ae_r, aer_l), (ae_i, aei_l),
                         (be_r, ber_l), (be_i, bei_l)):
            dst.append(src.reshape(_NP, 2 * l + 1, _B)
                       .transpose(1, 0, 2).reshape((2 * l + 1) * _NP, _B))
    aer = jnp.concatenate(aer_l, axis=0)
    aei = jnp.concatenate(aei_l, axis=0)
    ber = jnp.concatenate(ber_l, axis=0)
    bei = jnp.concatenate(bei_l, axis=0)
    wcat = jnp.concatenate([W0, W1, W2, W3, W4], axis=0)

    in_spec_ae = pl.BlockSpec((_AEROWS, _BT), lambda p, j: (0, j))
    out_shape = jax.ShapeDtypeStruct((_NT, _MTOT, _B), jnp.float32)
    out_spec = pl.BlockSpec((_NT, _MTOT, _BT), lambda p, j: (0, 0, j))
    outr, outi = pl.pallas_call(
        _cgbn_body,
        grid=(2, _NB),
        in_specs=[in_spec_ae, in_spec_ae, in_spec_ae, in_spec_ae,
                  pl.BlockSpec((_CTOT, _NT), lambda p, j: (0, 0))],
        out_specs=[out_spec, out_spec],
        out_shape=[out_shape, out_shape],
        scratch_shapes=[pltpu.VMEM((_CTOT, _BT), jnp.float32),
                        pltpu.VMEM((_CTOT, _NT), jnp.float32)],
    )(aer, aei, ber, bei, wcat)

    outs = []
    for l in range(_LMAX + 1):
        o_r = outr[:, _LMOFF[l]:_LMOFF[l] + 2 * l + 1, :]
        o_i = outi[:, _LMOFF[l]:_LMOFF[l] + 2 * l + 1, :]
        o = jnp.stack([o_r, o_i], axis=-1)
        outs.append(o.transpose(2, 0, 1, 3).reshape(_B, -1, 2))
    return jnp.concatenate(outs, axis=1)


# speedup vs baseline: 1.1460x; 1.1460x over previous
import functools
from math import factorial

import jax
import jax.numpy as jnp
from jax.experimental import pallas as pl
from jax.experimental.pallas import tpu as pltpu

_LMAX = 4
_NT = 8
_NP = _NT * _NT
_B = 1024
_BT = 128
_NB = _B // _BT

_LLLS = sorted([(l, l1, l2)
                for l1 in range(_LMAX + 1)
                for l2 in range(l1 + 1)
                for l in range(l1 - l2, min(l1 + l2, _LMAX) + 1)])
_TRIPLES_OF = [[(l1, l2) for (l, l1, l2) in _LLLS if l == lo]
               for lo in range(_LMAX + 1)]
_T_FF = [_NP * len(_TRIPLES_OF[lo]) for lo in range(_LMAX + 1)]
_CTOT = sum(_T_FF)
_CH_OFF = [sum(_T_FF[:l]) for l in range(_LMAX + 1)]
_LMOFF = [sum(2 * ll + 1 for ll in range(l)) for l in range(_LMAX + 1)]
_MTOT = _LMOFF[-1] + (2 * _LMAX + 1)
_AEOFF = [_NP * _LMOFF[l] for l in range(_LMAX + 1)]
_AEROWS = _NP * _MTOT


def _cgc(l1, m1, l2, m2, l, m):
    if m1 + m2 != m:
        return 0.0
    if l < abs(l1 - l2) or l > l1 + l2:
        return 0.0
    if abs(m1) > l1 or abs(m2) > l2 or abs(m) > l:
        return 0.0
    f = factorial
    pref = ((2 * l + 1) * f(l + l1 - l2) * f(l - l1 + l2) * f(l1 + l2 - l)
            / f(l1 + l2 + l + 1)) ** 0.5
    pref *= (f(l + m) * f(l - m) * f(l1 - m1) * f(l1 + m1)
             * f(l2 - m2) * f(l2 + m2)) ** 0.5
    s = 0.0
    kmin = max(0, l2 - l - m1, l1 + m2 - l)
    kmax = min(l1 + l2 - l, l1 - m1, l2 + m2)
    for k in range(kmin, kmax + 1):
        s += (-1.0) ** k / (f(k) * f(l1 + l2 - l - k) * f(l1 - m1 - k)
                            * f(l2 + m2 - k) * f(l - l2 + m1 + k)
                            * f(l - l1 - m2 + k))
    return pref * s


_PAIRS = sorted(set((l1, l2) for (_, l1, l2) in _LLLS))
_PLAN = {}
for (l1, l2) in _PAIRS:
    targets = {}
    for i1 in range(2 * l1 + 1):
        for i2 in range(2 * l2 + 1):
            m1, m2 = i1 - l1, i2 - l2
            tl = []
            for l in range(abs(l1 - l2), min(l1 + l2, _LMAX) + 1):
                if abs(m1 + m2) > l:
                    continue
                c = _cgc(l1, m1, l2, m2, l, m1 + m2)
                if c != 0.0:
                    t_idx = _TRIPLES_OF[l].index((l1, l2))
                    tl.append((l, t_idx, m1 + m2 + l, float(c)))
            if tl:
                targets[(i1, i2)] = tl
    _PLAN[(l1, l2)] = targets


def _compute_g(aer, aei, ber, bei):
    ar = {}
    ai = {}
    br = {}
    bi = {}
    for l in range(_LMAX + 1):
        for i in range(2 * l + 1):
            r0 = _AEOFF[l] + i * _NP
            ar[(l, i)] = aer[r0:r0 + _NP, :]
            ai[(l, i)] = aei[r0:r0 + _NP, :]
            br[(l, i)] = ber[r0:r0 + _NP, :]
            bi[(l, i)] = bei[r0:r0 + _NP, :]
    gr = {}
    gi = {}
    for (l1, l2) in _PAIRS:
        for (i1, i2), tl in _PLAN[(l1, l2)].items():
            xr, xi = ar[(l1, i1)], ai[(l1, i1)]
            yr, yi = br[(l2, i2)], bi[(l2, i2)]
            pr = xr * yr - xi * yi
            pi = xr * yi + xi * yr
            for (l, t_idx, m_idx, c) in tl:
                key = (l, t_idx, m_idx)
                if key in gr:
                    gr[key] = gr[key] + c * pr
                    gi[key] = gi[key] + c * pi
                else:
                    gr[key] = c * pr
                    gi[key] = c * pi
    zero = None
    for l in range(_LMAX + 1):
        for t in range(len(_TRIPLES_OF[l])):
            for m in range(2 * l + 1):
                if (l, t, m) not in gr:
                    if zero is None:
                        zero = jnp.zeros((_NP, _BT), jnp.float32)
                    gr[(l, t, m)] = zero
                    gi[(l, t, m)] = zero
    return gr, gi


def _cgbn_body(aer, aei, ber, bei, wcat, outr, outi, acc, wp):
    p = pl.program_id(0)
    j = pl.program_id(1)

    @pl.when(jnp.logical_and(p == 0, j == 0))
    def _():
        acc[...] = jnp.zeros_like(acc)

    gr, gi = _compute_g(aer[...], aei[...], ber[...], bei[...])

    @pl.when(p == 0)
    def _():
        for l in range(_LMAX + 1):
            for t in range(len(_TRIPLES_OF[l])):
                sq = None
                for m in range(2 * l + 1):
                    s = gr[(l, t, m)] * gr[(l, t, m)] \
                        + gi[(l, t, m)] * gi[(l, t, m)]
                    sq = s if sq is None else sq + s
                r0 = _CH_OFF[l] + t * _NP
                acc[r0:r0 + _NP, :] = acc[r0:r0 + _NP, :] + sq

    @pl.when(jnp.logical_and(p == 1, j == 0))
    def _():
        pw = jnp.sum(acc[...], axis=1, keepdims=True) * (1.0 / _B)
        wp[...] = wcat[...] * jax.lax.rsqrt(pw + 1e-5)

    @pl.when(p == 1)
    def _():
        for l in range(_LMAX + 1):
            nm = 2 * l + 1
            gmat_r = jnp.concatenate(
                [jnp.concatenate([gr[(l, t, m)] for m in range(nm)], axis=1)
                 for t in range(len(_TRIPLES_OF[l]))], axis=0)
            gmat_i = jnp.concatenate(
                [jnp.concatenate([gi[(l, t, m)] for m in range(nm)], axis=1)
                 for t in range(len(_TRIPLES_OF[l]))], axis=0)
            wl = wp[_CH_OFF[l]:_CH_OFF[l] + _T_FF[l], :]
            dn = (((0,), (0,)), ((), ()))
            o_r = jax.lax.dot_general(wl, gmat_r, dn,
                                      preferred_element_type=jnp.float32)
            o_i = jax.lax.dot_general(wl, gmat_i, dn,
                                      preferred_element_type=jnp.float32)
            for m in range(nm):
                outr[:, _LMOFF[l] + m, :] = o_r[:, m * _BT:(m + 1) * _BT]
                outi[:, _LMOFF[l] + m, :] = o_i[:, m * _BT:(m + 1) * _BT]


@functools.partial(jax.jit, static_argnums=())
def kernel(activations, W0, W1, W2, W3, W4):
    fr = activations[..., 0].T
    fi = activations[..., 1].T
    aer_l, aei_l, ber_l, bei_l = [], [], [], []
    off = 0
    for l in range(_LMAX + 1):
        n = _NT * (2 * l + 1)
        blk_r = fr[off:off + n, :].reshape(_NT, 2 * l + 1, _B)
        blk_i = fi[off:off + n, :].reshape(_NT, 2 * l + 1, _B)
        off += n
        ae_r = jnp.broadcast_to(blk_r[:, None], (_NT, _NT, 2 * l + 1, _B))
        ae_i = jnp.broadcast_to(blk_i[:, None], (_NT, _NT, 2 * l + 1, _B))
        be_r = jnp.broadcast_to(blk_r[None, :], (_NT, _NT, 2 * l + 1, _B))
        be_i = jnp.broadcast_to(blk_i[None, :], (_NT, _NT, 2 * l + 1, _B))
        for src, dst in ((ae_r, aer_l), (ae_i, aei_l),
                         (be_r, ber_l), (be_i, bei_l)):
            dst.append(src.reshape(_NP, 2 * l + 1, _B)
                       .transpose(1, 0, 2).reshape((2 * l + 1) * _NP, _B))
    aer = jnp.concatenate(aer_l, axis=0)
    aei = jnp.concatenate(aei_l, axis=0)
    ber = jnp.concatenate(ber_l, axis=0)
    bei = jnp.concatenate(bei_l, axis=0)
    wcat = jnp.concatenate([W0, W1, W2, W3, W4], axis=0)

    in_spec_ae = pl.BlockSpec((_AEROWS, _BT), lambda p, j: (0, j))
    out_shape = jax.ShapeDtypeStruct((_NT, _MTOT, _B), jnp.float32)
    out_spec = pl.BlockSpec((_NT, _MTOT, _BT), lambda p, j: (0, 0, j))
    outr, outi = pl.pallas_call(
        _cgbn_body,
        grid=(2, _NB),
        in_specs=[in_spec_ae, in_spec_ae, in_spec_ae, in_spec_ae,
                  pl.BlockSpec((_CTOT, _NT), lambda p, j: (0, 0))],
        out_specs=[out_spec, out_spec],
        out_shape=[out_shape, out_shape],
        scratch_shapes=[pltpu.VMEM((_CTOT, _BT), jnp.float32),
                        pltpu.VMEM((_CTOT, _NT), jnp.float32)],
    )(aer, aei, ber, bei, wcat)

    outs = []
    for l in range(_LMAX + 1):
        o_r = outr[:, _LMOFF[l]:_LMOFF[l] + 2 * l + 1, :]
        o_i = outi[:, _LMOFF[l]:_LMOFF[l] + 2 * l + 1, :]
        o = jnp.stack([o_r, o_i], axis=-1)
        outs.append(o.transpose(2, 0, 1, 3).reshape(_B, -1, 2))
    return jnp.concatenate(outs, axis=1)


# speedup vs baseline: 1.4424x; 1.2586x over previous
import functools
from math import factorial

import jax
import jax.numpy as jnp
from jax.experimental import pallas as pl
from jax.experimental.pallas import tpu as pltpu

_LMAX = 4
_NT = 8
_NP = _NT * _NT
_B = 1024
_BT = 128
_NB = _B // _BT

_LLLS = sorted([(l, l1, l2)
                for l1 in range(_LMAX + 1)
                for l2 in range(l1 + 1)
                for l in range(l1 - l2, min(l1 + l2, _LMAX) + 1)])
_TRIPLES_OF = [[(l1, l2) for (l, l1, l2) in _LLLS if l == lo]
               for lo in range(_LMAX + 1)]
_T_FF = [_NP * len(_TRIPLES_OF[lo]) for lo in range(_LMAX + 1)]
_CTOT = sum(_T_FF)
_CH_OFF = [sum(_T_FF[:l]) for l in range(_LMAX + 1)]
_LMOFF = [sum(2 * ll + 1 for ll in range(l)) for l in range(_LMAX + 1)]
_MTOT = _LMOFF[-1] + (2 * _LMAX + 1)
_AEOFF = [_NP * _LMOFF[l] for l in range(_LMAX + 1)]
_AEROWS = _NP * _MTOT


def _cgc(l1, m1, l2, m2, l, m):
    if m1 + m2 != m:
        return 0.0
    if l < abs(l1 - l2) or l > l1 + l2:
        return 0.0
    if abs(m1) > l1 or abs(m2) > l2 or abs(m) > l:
        return 0.0
    f = factorial
    pref = ((2 * l + 1) * f(l + l1 - l2) * f(l - l1 + l2) * f(l1 + l2 - l)
            / f(l1 + l2 + l + 1)) ** 0.5
    pref *= (f(l + m) * f(l - m) * f(l1 - m1) * f(l1 + m1)
             * f(l2 - m2) * f(l2 + m2)) ** 0.5
    s = 0.0
    kmin = max(0, l2 - l - m1, l1 + m2 - l)
    kmax = min(l1 + l2 - l, l1 - m1, l2 + m2)
    for k in range(kmin, kmax + 1):
        s += (-1.0) ** k / (f(k) * f(l1 + l2 - l - k) * f(l1 - m1 - k)
                            * f(l2 + m2 - k) * f(l - l2 + m1 + k)
                            * f(l - l1 - m2 + k))
    return pref * s


_PAIRS = sorted(set((l1, l2) for (_, l1, l2) in _LLLS))
_PLAN = {}
for (l1, l2) in _PAIRS:
    targets = {}
    for i1 in range(2 * l1 + 1):
        for i2 in range(2 * l2 + 1):
            m1, m2 = i1 - l1, i2 - l2
            tl = []
            for l in range(abs(l1 - l2), min(l1 + l2, _LMAX) + 1):
                if abs(m1 + m2) > l:
                    continue
                c = _cgc(l1, m1, l2, m2, l, m1 + m2)
                if c != 0.0:
                    t_idx = _TRIPLES_OF[l].index((l1, l2))
                    tl.append((l, t_idx, m1 + m2 + l, float(c)))
            if tl:
                targets[(i1, i2)] = tl
    _PLAN[(l1, l2)] = targets


def _compute_g(frm, fim):
    ar = {}
    ai = {}
    br = {}
    bi = {}
    for l in range(_LMAX + 1):
        for i in range(2 * l + 1):
            r0 = (_LMOFF[l] + i) * _NT
            a8r = frm[r0:r0 + _NT, :]
            a8i = fim[r0:r0 + _NT, :]
            ar[(l, i)] = jnp.broadcast_to(
                a8r[:, None, :], (_NT, _NT, _BT)).reshape(_NP, _BT)
            ai[(l, i)] = jnp.broadcast_to(
                a8i[:, None, :], (_NT, _NT, _BT)).reshape(_NP, _BT)
            br[(l, i)] = jnp.broadcast_to(
                a8r[None, :, :], (_NT, _NT, _BT)).reshape(_NP, _BT)
            bi[(l, i)] = jnp.broadcast_to(
                a8i[None, :, :], (_NT, _NT, _BT)).reshape(_NP, _BT)
    gr = {}
    gi = {}
    for (l1, l2) in _PAIRS:
        for (i1, i2), tl in _PLAN[(l1, l2)].items():
            xr, xi = ar[(l1, i1)], ai[(l1, i1)]
            yr, yi = br[(l2, i2)], bi[(l2, i2)]
            pr = xr * yr - xi * yi
            pi = xr * yi + xi * yr
            for (l, t_idx, m_idx, c) in tl:
                key = (l, t_idx, m_idx)
                if key in gr:
                    gr[key] = gr[key] + c * pr
                    gi[key] = gi[key] + c * pi
                else:
                    gr[key] = c * pr
                    gi[key] = c * pi
    zero = None
    for l in range(_LMAX + 1):
        for t in range(len(_TRIPLES_OF[l])):
            for m in range(2 * l + 1):
                if (l, t, m) not in gr:
                    if zero is None:
                        zero = jnp.zeros((_NP, _BT), jnp.float32)
                    gr[(l, t, m)] = zero
                    gi[(l, t, m)] = zero
    return gr, gi


def _cgbn_body(frm, fim, wcat, outr, outi, acc, wp):
    p = pl.program_id(0)
    j = pl.program_id(1)

    @pl.when(jnp.logical_and(p == 0, j == 0))
    def _():
        acc[...] = jnp.zeros_like(acc)

    gr, gi = _compute_g(frm[...], fim[...])

    @pl.when(p == 0)
    def _():
        for l in range(_LMAX + 1):
            for t in range(len(_TRIPLES_OF[l])):
                sq = None
                for m in range(2 * l + 1):
                    s = gr[(l, t, m)] * gr[(l, t, m)] \
                        + gi[(l, t, m)] * gi[(l, t, m)]
                    sq = s if sq is None else sq + s
                r0 = _CH_OFF[l] + t * _NP
                acc[r0:r0 + _NP, :] = acc[r0:r0 + _NP, :] + sq

    @pl.when(jnp.logical_and(p == 1, j == 0))
    def _():
        pw = jnp.sum(acc[...], axis=1, keepdims=True) * (1.0 / _B)
        wp[...] = wcat[...] * jax.lax.rsqrt(pw + 1e-5)

    @pl.when(p == 1)
    def _():
        for l in range(_LMAX + 1):
            nm = 2 * l + 1
            gmat_r = jnp.concatenate(
                [jnp.concatenate([gr[(l, t, m)] for m in range(nm)], axis=1)
                 for t in range(len(_TRIPLES_OF[l]))], axis=0)
            gmat_i = jnp.concatenate(
                [jnp.concatenate([gi[(l, t, m)] for m in range(nm)], axis=1)
                 for t in range(len(_TRIPLES_OF[l]))], axis=0)
            wl = wp[_CH_OFF[l]:_CH_OFF[l] + _T_FF[l], :]
            dn = (((0,), (0,)), ((), ()))
            o_r = jax.lax.dot_general(wl, gmat_r, dn,
                                      preferred_element_type=jnp.float32)
            o_i = jax.lax.dot_general(wl, gmat_i, dn,
                                      preferred_element_type=jnp.float32)
            for m in range(nm):
                outr[:, _LMOFF[l] + m, :] = o_r[:, m * _BT:(m + 1) * _BT]
                outi[:, _LMOFF[l] + m, :] = o_i[:, m * _BT:(m + 1) * _BT]


@functools.partial(jax.jit, static_argnums=())
def kernel(activations, W0, W1, W2, W3, W4):
    fr = activations[..., 0].T
    fi = activations[..., 1].T
    frm_l, fim_l = [], []
    off = 0
    for l in range(_LMAX + 1):
        n = _NT * (2 * l + 1)
        frm_l.append(fr[off:off + n, :].reshape(_NT, 2 * l + 1, _B)
                     .transpose(1, 0, 2).reshape(n, _B))
        fim_l.append(fi[off:off + n, :].reshape(_NT, 2 * l + 1, _B)
                     .transpose(1, 0, 2).reshape(n, _B))
        off += n
    frm = jnp.concatenate(frm_l, axis=0)
    fim = jnp.concatenate(fim_l, axis=0)
    wcat = jnp.concatenate([W0, W1, W2, W3, W4], axis=0)

    in_spec_f = pl.BlockSpec((_NT * _MTOT, _BT), lambda p, j: (0, j))
    out_shape = jax.ShapeDtypeStruct((_NT, _MTOT, _B), jnp.float32)
    out_spec = pl.BlockSpec((_NT, _MTOT, _BT), lambda p, j: (0, 0, j))
    outr, outi = pl.pallas_call(
        _cgbn_body,
        grid=(2, _NB),
        in_specs=[in_spec_f, in_spec_f,
                  pl.BlockSpec((_CTOT, _NT), lambda p, j: (0, 0))],
        out_specs=[out_spec, out_spec],
        out_shape=[out_shape, out_shape],
        scratch_shapes=[pltpu.VMEM((_CTOT, _BT), jnp.float32),
                        pltpu.VMEM((_CTOT, _NT), jnp.float32)],
    )(frm, fim, wcat)

    outs = []
    for l in range(_LMAX + 1):
        o_r = outr[:, _LMOFF[l]:_LMOFF[l] + 2 * l + 1, :]
        o_i = outi[:, _LMOFF[l]:_LMOFF[l] + 2 * l + 1, :]
        o = jnp.stack([o_r, o_i], axis=-1)
        outs.append(o.transpose(2, 0, 1, 3).reshape(_B, -1, 2))
    return jnp.concatenate(outs, axis=1)
